# Initial kernel scaffold; baseline (speedup 1.0000x reference)
import jax
import jax.numpy as jnp
from jax.experimental import pallas as pl


def kernel(loc, conf, iou, priors):
    raise NotImplementedError("write your pallas kernel here")


# speedup vs baseline: 12.6869x; 12.6869x over previous
import functools

import jax
import jax.numpy as jnp
from jax.experimental import pallas as pl
from jax.experimental.pallas import tpu as pltpu

_N = 20000
_TOP_K = 50
_IOU_THR = 0.3
_V0 = 0.1
_V1 = 0.2
_C = 128
_R = 160
_PAD = _R * _C
_NEG_INF = float('-inf')


def _nms_kernel(loc_ref, conf_ref, iou_ref, pri_ref, out_ref,
                s_ref, s0_ref, x1_ref, y1_ref, x2_ref, y2_ref, ar_ref,
                lm_ref):
    pcx = pri_ref[0]
    pcy = pri_ref[1]
    pw = pri_ref[2]
    ph = pri_ref[3]

    rowi = jax.lax.broadcasted_iota(jnp.int32, (_R, _C), 0)
    coli = jax.lax.broadcasted_iota(jnp.int32, (_R, _C), 1)
    lin = rowi * _C + coli

    cls = conf_ref[...]
    iouc = jnp.clip(iou_ref[...], 0.0, 1.0)
    scores = jnp.sqrt(cls * iouc)
    scores = jnp.where(lin < _N, scores, _NEG_INF)

    cx = pcx + loc_ref[0] * _V0 * pw
    cy = pcy + loc_ref[1] * _V0 * ph
    wx = pw * jnp.exp(loc_ref[2] * _V0) * 0.5
    hy = ph * jnp.exp(loc_ref[3] * _V1) * 0.5
    x1 = cx - wx
    y1 = cy - hy
    x2 = cx + wx
    y2 = cy + hy

    s_ref[...] = scores
    s0_ref[...] = scores
    x1_ref[...] = x1
    y1_ref[...] = y1
    x2_ref[...] = x2
    y2_ref[...] = y2
    ar_ref[...] = (x2 - x1) * (y2 - y1)
    for k in range(5):
        lm_ref[2 * k] = pcx + loc_ref[4 + 2 * k] * _V0 * pw
        lm_ref[2 * k + 1] = pcy + loc_ref[5 + 2 * k] * _V0 * ph

    coli1 = jax.lax.broadcasted_iota(jnp.int32, (1, _C), 1)

    def body(i, _):
        s = s_ref[...]
        m = jnp.max(s)
        idx = jnp.min(jnp.where(s == m, lin, _PAD))
        r = idx // _C
        c = idx % _C

        def ext(ref2d):
            row = ref2d[pl.ds(r, 1), :]
            return jnp.sum(jnp.where(coli1 == c, row, 0.0))

        bx1 = ext(x1_ref)
        by1 = ext(y1_ref)
        bx2 = ext(x2_ref)
        by2 = ext(y2_ref)
        barea = ext(ar_ref)
        bscore = ext(s0_ref)

        vals = [bx1, by1, bx2, by2]
        for k in range(10):
            row = lm_ref[k, pl.ds(r, 1), :]
            vals.append(jnp.sum(jnp.where(coli1 == c, row, 0.0)))
        vals.append(bscore)

        out_row = jnp.zeros((1, _C), jnp.float32)
        for j, v in enumerate(vals):
            out_row = jnp.where(coli1 == j, v, out_row)
        out_ref[pl.ds(i, 1), :] = out_row

        ix1 = jnp.maximum(bx1, x1_ref[...])
        iy1 = jnp.maximum(by1, y1_ref[...])
        ix2 = jnp.minimum(bx2, x2_ref[...])
        iy2 = jnp.minimum(by2, y2_ref[...])
        inter = (jnp.maximum(ix2 - ix1, 0.0)
                 * jnp.maximum(iy2 - iy1, 0.0))
        union = barea + ar_ref[...] - inter
        iouv = inter / jnp.maximum(union, 1e-12)
        s_ref[...] = jnp.where(iouv <= _IOU_THR, s_ref[...], _NEG_INF)
        return 0

    jax.lax.fori_loop(0, _TOP_K, body, 0)


def _plane(x):
    return jnp.pad(x, (0, _PAD - _N)).reshape(_R, _C)


@jax.jit
def kernel(loc, conf, iou, priors):
    loc_p = jnp.stack([_plane(loc[:, k]) for k in range(14)])
    conf_p = _plane(conf[:, 1])
    iou_p = _plane(iou[:, 0])
    pri_p = jnp.stack([_plane(priors[:, k]) for k in range(4)])

    out = pl.pallas_call(
        _nms_kernel,
        out_shape=jax.ShapeDtypeStruct((_TOP_K, _C), jnp.float32),
        scratch_shapes=[
            pltpu.VMEM((_R, _C), jnp.float32),
            pltpu.VMEM((_R, _C), jnp.float32),
            pltpu.VMEM((_R, _C), jnp.float32),
            pltpu.VMEM((_R, _C), jnp.float32),
            pltpu.VMEM((_R, _C), jnp.float32),
            pltpu.VMEM((_R, _C), jnp.float32),
            pltpu.VMEM((_R, _C), jnp.float32),
            pltpu.VMEM((10, _R, _C), jnp.float32),
        ],
    )(loc_p, conf_p, iou_p, pri_p)
    return out[:, :15]
